# baseline (speedup 1.0000x reference)
import jax
import jax.numpy as jnp
from jax.experimental import pallas as pl
from jax.experimental.pallas import tpu as pltpu
from jax.experimental.pallas import tpu_sc as plsc

_DATA_VARIANCE = 0.06327
_COMMITMENT_COST = 0.25


def _s2d(x):
    B, H, W, C = x.shape
    x = x.reshape(B, H // 2, 2, W // 2, 2, C)
    x = x.transpose(0, 1, 3, 2, 4, 5)
    return x.reshape(B, H // 2, W // 2, 4 * C)


def _d2s(y):
    B, h, w, C4 = y.shape
    C = C4 // 4
    y = y.reshape(B, h, w, 2, 2, C)
    y = y.transpose(0, 1, 3, 2, 4, 5)
    return y.reshape(B, 2 * h, 2 * w, C)


def _pad1(x):
    return jnp.pad(x, ((0, 0), (1, 1), (1, 1), (0, 0)))


def _conv_taps(xp, w_stack, bias, taps, oh, ow, relu, post_w=None, post_b=None):
    B, Hp, Wp, Cin = xp.shape
    T, _, Cout = w_stack.shape
    C2 = Cout if post_w is None else post_w.shape[1]

    def body(*refs):
        if post_w is None:
            x_ref, w_ref, b_ref, o_ref = refs
        else:
            x_ref, w_ref, b_ref, pw_ref, pb_ref, o_ref = refs
        acc = None
        for t, (dy, dx) in enumerate(taps):
            xs = x_ref[0, dy:dy + oh, dx:dx + ow, :].reshape(oh * ow, Cin)
            p = jnp.dot(xs, w_ref[t], preferred_element_type=jnp.float32)
            acc = p if acc is None else acc + p
        y = acc + b_ref[...]
        if relu:
            y = jnp.maximum(y, 0.0)
        if post_w is not None:
            y = jnp.dot(y, pw_ref[...], preferred_element_type=jnp.float32) + pb_ref[...]
        o_ref[0] = y.reshape(oh, ow, C2)

    in_specs = [
        pl.BlockSpec((1, Hp, Wp, Cin), lambda i: (i, 0, 0, 0)),
        pl.BlockSpec((T, Cin, Cout), lambda i: (0, 0, 0)),
        pl.BlockSpec((1, Cout), lambda i: (0, 0)),
    ]
    args = [xp, w_stack, bias]
    if post_w is not None:
        in_specs += [pl.BlockSpec(post_w.shape, lambda i: (0, 0)),
                     pl.BlockSpec(post_b.shape, lambda i: (0, 0))]
        args += [post_w, post_b]
    return pl.pallas_call(
        body,
        grid=(B,),
        in_specs=in_specs,
        out_specs=pl.BlockSpec((1, oh, ow, C2), lambda i: (i, 0, 0, 0)),
        out_shape=jax.ShapeDtypeStruct((B, oh, ow, C2), jnp.float32),
    )(*args)


def _subpix_taps(w):
    Cin, Cout = w.shape[2], w.shape[3]
    wz = w.reshape(2, 2, 2, 2, Cin, Cout)
    wz = wz.transpose(1, 3, 0, 2, 4, 5)
    return wz.reshape(4, 4, Cin, Cout)


_GROUPS = ((0, 0), (0, 1), (1, 0), (1, 1))


def _conv_transpose_subpix(xp, wg, bias, n, relu):
    B, Hp, Wp, Cin = xp.shape
    Cout = wg.shape[3]

    def body(x_ref, w_ref, b_ref, o_ref):
        for g, (r, s) in enumerate(_GROUPS):
            acc = None
            for t, (a, b2) in enumerate(_GROUPS):
                xs = x_ref[0, a + r:a + r + n, b2 + s:b2 + s + n, :]
                xs = xs.reshape(n * n, Cin)
                p = jnp.dot(xs, w_ref[g, t], preferred_element_type=jnp.float32)
                acc = p if acc is None else acc + p
            y = acc + b_ref[...]
            if relu:
                y = jnp.maximum(y, 0.0)
            o_ref[0, :, :, g * Cout:(g + 1) * Cout] = y.reshape(n, n, Cout)

    return pl.pallas_call(
        body,
        grid=(B,),
        in_specs=[
            pl.BlockSpec((1, Hp, Wp, Cin), lambda i: (i, 0, 0, 0)),
            pl.BlockSpec((4, 4, Cin, Cout), lambda i: (0, 0, 0, 0)),
            pl.BlockSpec((1, Cout), lambda i: (0, 0)),
        ],
        out_specs=pl.BlockSpec((1, n, n, 4 * Cout), lambda i: (i, 0, 0, 0)),
        out_shape=jax.ShapeDtypeStruct((B, n, n, 4 * Cout), jnp.float32),
    )(xp, wg, bias)


def _conv_transpose_final(xp, wg, bias, x_s2d, n, th):
    B, Hp, Wp, Cin = xp.shape
    Cout = wg.shape[3]
    S = n // th

    def body(x_ref, w_ref, b_ref, t_ref, o_ref, r_ref):
        b = pl.program_id(0)
        s = pl.program_id(1)

        @pl.when(jnp.logical_and(b == 0, s == 0))
        def _():
            r_ref[...] = jnp.zeros((1, 1), jnp.float32)

        base = s * th
        part = None
        for g, (r, s2) in enumerate(_GROUPS):
            acc = None
            for t, (a, b2) in enumerate(_GROUPS):
                xs = x_ref[0, pl.ds(base + a + r, th), b2 + s2:b2 + s2 + n, :]
                xs = xs.reshape(th * n, Cin)
                p = jnp.dot(xs, w_ref[g, t], preferred_element_type=jnp.float32)
                acc = p if acc is None else acc + p
            y = acc + b_ref[...]
            o_ref[0, :, :, g * Cout:(g + 1) * Cout] = y.reshape(th, n, Cout)
            tgt = t_ref[0, :, :, g * Cout:(g + 1) * Cout].reshape(th * n, Cout)
            d = y - tgt
            sq = jnp.sum(d * d)
            part = sq if part is None else part + sq
        r_ref[...] += part.reshape(1, 1)

    return pl.pallas_call(
        body,
        grid=(B, S),
        in_specs=[
            pl.BlockSpec((1, Hp, Wp, Cin), lambda i, j: (i, 0, 0, 0)),
            pl.BlockSpec((4, 4, Cin, Cout), lambda i, j: (0, 0, 0, 0)),
            pl.BlockSpec((1, Cout), lambda i, j: (0, 0)),
            pl.BlockSpec((1, th, n, 4 * Cout), lambda i, j: (i, j, 0, 0)),
        ],
        out_specs=[
            pl.BlockSpec((1, th, n, 4 * Cout), lambda i, j: (i, j, 0, 0)),
            pl.BlockSpec((1, 1), lambda i, j: (0, 0)),
        ],
        out_shape=[
            jax.ShapeDtypeStruct((B, n, n, 4 * Cout), jnp.float32),
            jax.ShapeDtypeStruct((1, 1), jnp.float32),
        ],
    )(xp, wg, bias, x_s2d)


_VQ_ROWS = 512


def _vq_argmin(zf, cbT):
    N, D = zf.shape
    K = cbT.shape[1]
    R = _VQ_ROWS
    G = N // R

    def body(z_ref, c_ref, i_ref, d_ref):
        step = pl.program_id(0)

        @pl.when(step == 0)
        def _():
            d_ref[...] = jnp.zeros((1, 1), jnp.float32)

        zt = z_ref[...]
        cbt = c_ref[...]
        cross = jnp.dot(zt, cbt, preferred_element_type=jnp.float32)
        cn = jnp.sum(cbt * cbt, axis=0, keepdims=True)
        zn = jnp.sum(zt * zt, axis=1, keepdims=True)
        dist = zn - 2.0 * cross + cn
        m = jnp.min(dist, axis=1, keepdims=True)
        iota = jax.lax.broadcasted_iota(jnp.int32, (R, K), 1)
        idx = jnp.min(jnp.where(dist == m, iota, K), axis=1)
        i_ref[0, 0, :] = idx
        d_ref[...] += jnp.sum(m).reshape(1, 1)

    return pl.pallas_call(
        body,
        grid=(G,),
        in_specs=[
            pl.BlockSpec((R, D), lambda i: (i, 0)),
            pl.BlockSpec((D, K), lambda i: (0, 0)),
        ],
        out_specs=[
            pl.BlockSpec((1, 1, R), lambda i: (i, 0, 0)),
            pl.BlockSpec((1, 1), lambda i: (0, 0)),
        ],
        out_shape=[
            jax.ShapeDtypeStruct((G, 1, R), jnp.int32),
            jax.ShapeDtypeStruct((1, 1), jnp.float32),
        ],
    )(zf, cbT)


_GATHER_WINDOW = 128


def _sc_gather(codebook, idx_flat):
    N = idx_flat.shape[0]
    D = codebook.shape[1]
    W = _GATHER_WINDOW
    idx2 = idx_flat.reshape(1, N)
    mesh = plsc.VectorSubcoreMesh(core_axis_name="core", subcore_axis_name="subcore")

    @pl.kernel(out_type=jax.ShapeDtypeStruct((N, D), codebook.dtype), mesh=mesh)
    def k(cb_hbm, i_hbm, o_hbm):
        def body(i_vmem, o_vmem):
            pltpu.sync_copy(cb_hbm.at[i_vmem.at[0]], o_vmem)

        pltpu.emit_pipeline(
            body,
            grid=(N // W,),
            in_specs=[pl.BlockSpec((1, W), index_map=lambda i: (0, i))],
            out_specs=[pl.BlockSpec((W, D), index_map=lambda i: (i, 0))],
            core_axis_name=("core", "subcore"),
            dimension_semantics=(pltpu.PARALLEL,),
        )(i_hbm, o_hbm)

    return k(codebook, idx2)


def _stride2_taps(w):
    Cin, Cout = w.shape[2], w.shape[3]
    wz = w.reshape(2, 2, 2, 2, Cin, Cout)
    wz = wz.transpose(0, 2, 1, 3, 4, 5)
    return wz.reshape(4, 4 * Cin, Cout)


_TAPS2 = ((0, 0), (0, 1), (1, 0), (1, 1))
_TAPS3 = tuple((dy, dx) for dy in range(3) for dx in range(3))


def kernel(inputs, enc_w1, enc_b1, enc_w2, enc_b2, enc_w3, enc_b3, pre_vq_w,
           pre_vq_b, codebook, dec_w1, dec_b1, dec_wt1, dec_bt1, dec_wt2,
           dec_bt2, is_training):
    B = inputs.shape[0]

    x1 = _s2d(_pad1(inputs))
    h1 = _conv_taps(x1, _stride2_taps(enc_w1), enc_b1.reshape(1, -1),
                    _TAPS2, 112, 112, relu=True)

    x2 = _s2d(_pad1(h1))
    h2 = _conv_taps(x2, _stride2_taps(enc_w2), enc_b2.reshape(1, -1),
                    _TAPS2, 56, 56, relu=True)

    z = _conv_taps(_pad1(h2), enc_w3.reshape(9, 128, 128),
                   enc_b3.reshape(1, -1), _TAPS3, 56, 56, relu=True,
                   post_w=pre_vq_w.reshape(128, 64), post_b=pre_vq_b.reshape(1, 64))

    N = B * 56 * 56
    zf = z.reshape(N, 64)
    idx, dist_sum = _vq_argmin(zf, codebook.T)
    cb_pad = jnp.pad(codebook, ((0, 0), (0, 64)))
    quant = _sc_gather(cb_pad, idx.reshape(N)).reshape(B, 56, 56, 128)
    vq_loss = (1.0 + _COMMITMENT_COST) * dist_sum[0, 0] / (N * 64)

    w1p = jnp.pad(dec_w1.reshape(9, 64, 128), ((0, 0), (0, 64), (0, 0)))
    d1 = _conv_taps(_pad1(quant), w1p,
                    dec_b1.reshape(1, -1), _TAPS3, 56, 56, relu=True)

    d2 = _conv_transpose_subpix(_pad1(d1), _subpix_taps(dec_wt1),
                                dec_bt1.reshape(1, -1), 56, relu=True)
    d2 = _d2s(d2)

    x_s2d = _s2d(inputs)
    y, rsum = _conv_transpose_final(_pad1(d2), _subpix_taps(dec_wt2),
                                    dec_bt2.reshape(1, -1), x_s2d, 112, 28)
    x_recon = _d2s(y)

    recon_error = rsum[0, 0] / (B * 224 * 224 * 3) / _DATA_VARIANCE
    loss = recon_error + vq_loss
    return (z, x_recon, loss, recon_error, vq_loss)


# speedup vs baseline: 1.1822x; 1.1822x over previous
import jax
import jax.numpy as jnp
from jax.experimental import pallas as pl
from jax.experimental.pallas import tpu as pltpu
from jax.experimental.pallas import tpu_sc as plsc

_DATA_VARIANCE = 0.06327
_COMMITMENT_COST = 0.25

_GROUPS = ((0, 0), (0, 1), (1, 0), (1, 1))
_TAPS2 = ((0, 0), (0, 1), (1, 0), (1, 1))
_TAPS3 = tuple((dy, dx) for dy in range(3) for dx in range(3))


def _s2d(x):
    B, H, W, C = x.shape
    x = x.reshape(B, H // 2, 2, W // 2, 2, C)
    x = x.transpose(0, 1, 3, 2, 4, 5)
    return x.reshape(B, H // 2, W // 2, 4 * C)


def _d2s(y):
    B, h, w, C4 = y.shape
    C = C4 // 4
    y = y.reshape(B, h, w, 2, 2, C)
    y = y.transpose(0, 1, 3, 2, 4, 5)
    return y.reshape(B, 2 * h, 2 * w, C)


def _pad1(x):
    return jnp.pad(x, ((0, 0), (1, 1), (1, 1), (0, 0)))


def _dot_taps(x_view, w_ref, taps, oh, ow, cin):
    acc = None
    for t, (dy, dx) in enumerate(taps):
        xs = x_view(dy, dx).reshape(oh * ow, cin)
        p = jnp.dot(xs, w_ref[t * cin:(t + 1) * cin, :],
                    preferred_element_type=jnp.float32)
        acc = p if acc is None else acc + p
    return acc


def _conv_layer(x, w_flat, bias, taps, oh, ow, relu, *, post_w=None, post_b=None,
                stage_pad=False, out_kind="raw"):
    B = x.shape[0]
    Cin = x.shape[3]
    Cout = w_flat.shape[1]
    C2 = Cout if post_w is None else post_w.shape[1]
    if out_kind == "raw":
        out_sds = jax.ShapeDtypeStruct((B, oh, ow, C2), jnp.float32)
    elif out_kind == "pad":
        out_sds = jax.ShapeDtypeStruct((B, oh + 2, ow + 2, C2), jnp.float32)
    else:
        out_sds = jax.ShapeDtypeStruct((B, oh // 2 + 1, ow // 2 + 1, 4 * C2),
                                       jnp.float32)

    def body(*refs):
        if post_w is None:
            if stage_pad:
                x_ref, w_ref, b_ref, o_ref, s_ref = refs
            else:
                x_ref, w_ref, b_ref, o_ref = refs
        else:
            x_ref, w_ref, b_ref, pw_ref, pb_ref, o_ref = refs
        if stage_pad:
            s_ref[...] = jnp.zeros(s_ref.shape, jnp.float32)
            s_ref[1:oh + 1, 1:ow + 1, :] = x_ref[0]
            x_view = lambda dy, dx: s_ref[dy:dy + oh, dx:dx + ow, :]
        else:
            x_view = lambda dy, dx: x_ref[0, dy:dy + oh, dx:dx + ow, :]
        y = _dot_taps(x_view, w_ref, taps, oh, ow, Cin)
        y = y + b_ref[...]
        if relu:
            y = jnp.maximum(y, 0.0)
        if post_w is not None:
            y = jnp.dot(y, pw_ref[...], preferred_element_type=jnp.float32) + pb_ref[...]
        if out_kind == "raw":
            o_ref[0] = y.reshape(oh, ow, C2)
        elif out_kind == "pad":
            o_ref[...] = jnp.zeros(o_ref.shape, jnp.float32)
            o_ref[0, 1:oh + 1, 1:ow + 1, :] = y.reshape(oh, ow, C2)
        else:
            h, w2 = oh // 2, ow // 2
            yi = y.reshape(h, 2, w2, 2, C2)
            o_ref[...] = jnp.zeros(o_ref.shape, jnp.float32)
            o_ref[0, 1:h + 1, 1:w2 + 1, 0 * C2:1 * C2] = yi[:, 1, :, 1, :]
            o_ref[0, 1:h + 1, 0:w2, 1 * C2:2 * C2] = yi[:, 1, :, 0, :]
            o_ref[0, 0:h, 1:w2 + 1, 2 * C2:3 * C2] = yi[:, 0, :, 1, :]
            o_ref[0, 0:h, 0:w2, 3 * C2:4 * C2] = yi[:, 0, :, 0, :]

    in_specs = [
        pl.BlockSpec((1,) + x.shape[1:], lambda i: (i, 0, 0, 0)),
        pl.BlockSpec(w_flat.shape, lambda i: (0, 0)),
        pl.BlockSpec(bias.shape, lambda i: (0, 0)),
    ]
    args = [x, w_flat, bias]
    if post_w is not None:
        in_specs += [pl.BlockSpec(post_w.shape, lambda i: (0, 0)),
                     pl.BlockSpec(post_b.shape, lambda i: (0, 0))]
        args += [post_w, post_b]
    scratch = []
    if stage_pad:
        scratch = [pltpu.VMEM((oh + 2, ow + 2, Cin), jnp.float32)]
    return pl.pallas_call(
        body,
        grid=(B,),
        in_specs=in_specs,
        out_specs=pl.BlockSpec((1,) + out_sds.shape[1:], lambda i: (i, 0, 0, 0)),
        out_shape=out_sds,
        scratch_shapes=scratch,
    )(*args)


def _subpix_taps(w):
    Cin, Cout = w.shape[2], w.shape[3]
    wz = w.reshape(2, 2, 2, 2, Cin, Cout)
    wz = wz.transpose(1, 3, 0, 2, 4, 5)
    return wz.reshape(4, 4 * Cin, Cout)


def _conv_transpose_subpix(xp, wg, bias, n, relu):
    B, Hp, Wp, Cin = xp.shape
    Cout = wg.shape[2]

    def body(x_ref, w_ref, b_ref, o_ref):
        for g, (r, s) in enumerate(_GROUPS):
            view = lambda a, b2: x_ref[0, a + r:a + r + n, b2 + s:b2 + s + n, :]
            y = _dot_taps(view, w_ref.at[g], _TAPS2, n, n, Cin)
            y = y + b_ref[...]
            if relu:
                y = jnp.maximum(y, 0.0)
            o_ref[0, :, :, g * Cout:(g + 1) * Cout] = y.reshape(n, n, Cout)

    return pl.pallas_call(
        body,
        grid=(B,),
        in_specs=[
            pl.BlockSpec((1, Hp, Wp, Cin), lambda i: (i, 0, 0, 0)),
            pl.BlockSpec((4, 4 * Cin, Cout), lambda i: (0, 0, 0)),
            pl.BlockSpec((1, Cout), lambda i: (0, 0)),
        ],
        out_specs=pl.BlockSpec((1, n, n, 4 * Cout), lambda i: (i, 0, 0, 0)),
        out_shape=jax.ShapeDtypeStruct((B, n, n, 4 * Cout), jnp.float32),
    )(xp, wg, bias)


def _conv_transpose_final(xp, wg, bias, x_s2d, n, th):
    B, Hp, Wp, Cin = xp.shape
    Cout = wg.shape[2]
    S = n // th

    def body(x_ref, w_ref, b_ref, t_ref, o_ref, r_ref):
        b = pl.program_id(0)
        s = pl.program_id(1)

        @pl.when(jnp.logical_and(b == 0, s == 0))
        def _():
            r_ref[...] = jnp.zeros((1, 1), jnp.float32)

        base = s * th
        part = None
        for g, (r, s2) in enumerate(_GROUPS):
            view = lambda a, b2: x_ref[0, pl.ds(base + a + r, th),
                                       b2 + s2:b2 + s2 + n, :]
            y = _dot_taps(view, w_ref.at[g], _TAPS2, th, n, Cin)
            y = y + b_ref[...]
            o_ref[0, :, :, g * Cout:(g + 1) * Cout] = y.reshape(th, n, Cout)
            tgt = t_ref[0, :, :, g * Cout:(g + 1) * Cout].reshape(th * n, Cout)
            d = y - tgt
            sq = jnp.sum(d * d)
            part = sq if part is None else part + sq
        r_ref[...] += part.reshape(1, 1)

    return pl.pallas_call(
        body,
        grid=(B, S),
        in_specs=[
            pl.BlockSpec((1, Hp, Wp, Cin), lambda i, j: (i, 0, 0, 0)),
            pl.BlockSpec((4, 4 * Cin, Cout), lambda i, j: (0, 0, 0)),
            pl.BlockSpec((1, Cout), lambda i, j: (0, 0)),
            pl.BlockSpec((1, th, n, 4 * Cout), lambda i, j: (i, j, 0, 0)),
        ],
        out_specs=[
            pl.BlockSpec((1, th, n, 4 * Cout), lambda i, j: (i, j, 0, 0)),
            pl.BlockSpec((1, 1), lambda i, j: (0, 0)),
        ],
        out_shape=[
            jax.ShapeDtypeStruct((B, n, n, 4 * Cout), jnp.float32),
            jax.ShapeDtypeStruct((1, 1), jnp.float32),
        ],
    )(xp, wg, bias, x_s2d)


_VQ_ROWS = 512


def _vq_argmin(zf, cbT):
    N, D = zf.shape
    K = cbT.shape[1]
    R = _VQ_ROWS
    G = N // R

    def body(z_ref, c_ref, i_ref, d_ref):
        step = pl.program_id(0)

        @pl.when(step == 0)
        def _():
            d_ref[...] = jnp.zeros((1, 1), jnp.float32)

        zt = z_ref[...]
        cbt = c_ref[...]
        cross = jnp.dot(zt, cbt, preferred_element_type=jnp.float32)
        cn = jnp.sum(cbt * cbt, axis=0, keepdims=True)
        zn = jnp.sum(zt * zt, axis=1, keepdims=True)
        dist = zn - 2.0 * cross + cn
        m = jnp.min(dist, axis=1, keepdims=True)
        iota = jax.lax.broadcasted_iota(jnp.int32, (R, K), 1)
        idx = jnp.min(jnp.where(dist == m, iota, K), axis=1)
        i_ref[0, 0, :] = idx
        d_ref[...] += jnp.sum(m).reshape(1, 1)

    return pl.pallas_call(
        body,
        grid=(G,),
        in_specs=[
            pl.BlockSpec((R, D), lambda i: (i, 0)),
            pl.BlockSpec((D, K), lambda i: (0, 0)),
        ],
        out_specs=[
            pl.BlockSpec((1, 1, R), lambda i: (i, 0, 0)),
            pl.BlockSpec((1, 1), lambda i: (0, 0)),
        ],
        out_shape=[
            jax.ShapeDtypeStruct((G, 1, R), jnp.int32),
            jax.ShapeDtypeStruct((1, 1), jnp.float32),
        ],
    )(zf, cbT)


_GATHER_WINDOW = 256


def _sc_gather(codebook, idx_flat):
    N = idx_flat.shape[0]
    D = codebook.shape[1]
    W = _GATHER_WINDOW
    idx2 = idx_flat.reshape(1, N)
    mesh = plsc.VectorSubcoreMesh(core_axis_name="core", subcore_axis_name="subcore")

    @pl.kernel(out_type=jax.ShapeDtypeStruct((N, D), codebook.dtype), mesh=mesh)
    def k(cb_hbm, i_hbm, o_hbm):
        def body(i_vmem, o_vmem):
            pltpu.sync_copy(cb_hbm.at[i_vmem.at[0]], o_vmem)

        pltpu.emit_pipeline(
            body,
            grid=(N // W,),
            in_specs=[pl.BlockSpec((1, W), index_map=lambda i: (0, i))],
            out_specs=[pl.BlockSpec((W, D), index_map=lambda i: (i, 0))],
            core_axis_name=("core", "subcore"),
            dimension_semantics=(pltpu.PARALLEL,),
        )(i_hbm, o_hbm)

    return k(codebook, idx2)


def _stride2_taps(w):
    Cin, Cout = w.shape[2], w.shape[3]
    wz = w.reshape(2, 2, 2, 2, Cin, Cout)
    wz = wz.transpose(0, 2, 1, 3, 4, 5)
    return wz.reshape(16 * Cin, Cout)


def kernel(inputs, enc_w1, enc_b1, enc_w2, enc_b2, enc_w3, enc_b3, pre_vq_w,
           pre_vq_b, codebook, dec_w1, dec_b1, dec_wt1, dec_bt1, dec_wt2,
           dec_bt2, is_training):
    B = inputs.shape[0]

    x1 = _s2d(_pad1(inputs))
    h1p = _conv_layer(x1, _stride2_taps(enc_w1), enc_b1.reshape(1, -1),
                      _TAPS2, 112, 112, relu=True, out_kind="s2dpad")

    h2p = _conv_layer(h1p, _stride2_taps(enc_w2), enc_b2.reshape(1, -1),
                      _TAPS2, 56, 56, relu=True, out_kind="pad")

    z = _conv_layer(h2p, enc_w3.reshape(9 * 128, 128), enc_b3.reshape(1, -1),
                    _TAPS3, 56, 56, relu=True,
                    post_w=pre_vq_w.reshape(128, 64),
                    post_b=pre_vq_b.reshape(1, 64))

    N = B * 56 * 56
    zf = z.reshape(N, 64)
    idx, dist_sum = _vq_argmin(zf, codebook.T)
    cb_pad = jnp.pad(codebook, ((0, 0), (0, 64)))
    quant = _sc_gather(cb_pad, idx.reshape(N)).reshape(B, 56, 56, 128)
    vq_loss = (1.0 + _COMMITMENT_COST) * dist_sum[0, 0] / (N * 64)

    w1p = jnp.pad(dec_w1, ((0, 0), (0, 0), (0, 64), (0, 0)))
    d1p = _conv_layer(quant, w1p.reshape(9 * 128, 128), dec_b1.reshape(1, -1),
                      _TAPS3, 56, 56, relu=True, stage_pad=True, out_kind="pad")

    d2 = _conv_transpose_subpix(d1p, _subpix_taps(dec_wt1),
                                dec_bt1.reshape(1, -1), 56, relu=True)
    d2 = _d2s(d2)

    x_s2d = _s2d(inputs)
    y, rsum = _conv_transpose_final(_pad1(d2), _subpix_taps(dec_wt2),
                                    dec_bt2.reshape(1, -1), x_s2d, 112, 28)
    x_recon = _d2s(y)

    recon_error = rsum[0, 0] / (B * 224 * 224 * 3) / _DATA_VARIANCE
    loss = recon_error + vq_loss
    return (z, x_recon, loss, recon_error, vq_loss)
